# baseline (speedup 1.0000x reference)
import functools

import jax
import jax.numpy as jnp
from jax import lax
from jax.experimental import pallas as pl
from jax.experimental.pallas import tpu as pltpu
from jax.experimental.pallas import tpu_sc as plsc

B, N, K, DP, DM = 2, 2048, 16, 128, 256
BN = B * N

def _pack128(a):
    lo = a[:, :128].astype(jnp.bfloat16).astype(jnp.float32)
    hi = a[:, 128:].astype(jnp.bfloat16).astype(jnp.float32)
    lo_i = lax.bitcast_convert_type(lo, jnp.int32)
    hi_i = lax.bitcast_convert_type(hi, jnp.int32)
    return lax.shift_right_logical(lo_i, 16) | (hi_i & jnp.int32(-65536))


def _unpack128(xi):
    lo = lax.bitcast_convert_type(lax.shift_left(xi, 16), jnp.float32)
    hi = lax.bitcast_convert_type(xi & jnp.int32(-65536), jnp.float32)
    return jnp.concatenate([lo, hi], axis=1)


_MQ = 512


def _knn_body(xyz_ref, xyzT_ref, feat_ref, fc1w_ref, fc1b_ref,
              wq_ref, wk_ref, wv_ref, xyz3_ref, d1_ref,
              idx_ref, q_ref, tab_ref, p_ref):
    x = jnp.dot(feat_ref[...], fc1w_ref[...],
                preferred_element_type=jnp.float32) + fc1b_ref[...]
    xb = x.astype(jnp.bfloat16)
    q_ref[...] = jnp.dot(xb, wq_ref[...].astype(jnp.bfloat16),
                         preferred_element_type=jnp.float32)
    xk = jnp.dot(xb, wk_ref[...].astype(jnp.bfloat16),
                 preferred_element_type=jnp.float32)
    xv = jnp.dot(xb, wv_ref[...].astype(jnp.bfloat16),
                 preferred_element_type=jnp.float32)
    p = jnp.dot(xyz3_ref[...], d1_ref[...],
                preferred_element_type=jnp.float32)
    p_ref[...] = p
    tab_ref[...] = jnp.concatenate(
        [_pack128(xk), _pack128(xv), _pack128(p)], axis=1)
    xq = xyz_ref[0]
    xa = xyzT_ref[0]
    d0 = xq[:, 0:1] - xa[0:1, :]
    d1 = xq[:, 1:2] - xa[1:2, :]
    d2 = xq[:, 2:3] - xa[2:3, :]
    D = d0 * d0 + d1 * d1 + d2 * d2
    iota = lax.broadcasted_iota(jnp.int32, (_MQ, N), 1).astype(jnp.float32)
    cols = []
    for _ in range(K):
        mval = jnp.min(D, axis=1, keepdims=True)
        idxk = jnp.min(jnp.where(D == mval, iota, jnp.float32(N)),
                       axis=1, keepdims=True)
        cols.append(idxk)
        D = jnp.where(iota == idxk, jnp.float32(jnp.inf), D)
    idx_ref[...] = jnp.concatenate(cols, axis=1).astype(jnp.int32)


def _knn(xyzb, xyzTb, featb, fc1_w, fc1_b, wq, wk, wv, xyz3b, d1_w):
    nb = N // _MQ
    full = lambda m: (0, 0)
    row = lambda m: (m, 0)
    return pl.pallas_call(
        _knn_body,
        grid=(nb,),
        in_specs=[
            pl.BlockSpec((1, _MQ, 3), lambda m: (0, m, 0)),
            pl.BlockSpec((1, 3, N), lambda m: (0, 0, 0)),
            pl.BlockSpec((_MQ, DP), row),
            pl.BlockSpec((DP, DM), full),
            pl.BlockSpec((1, DM), full),
            pl.BlockSpec((DM, DM), full),
            pl.BlockSpec((DM, DM), full),
            pl.BlockSpec((DM, DM), full),
            pl.BlockSpec((_MQ, 3), row),
            pl.BlockSpec((3, DM), full),
        ],
        out_specs=[pl.BlockSpec((_MQ, K), row),
                   pl.BlockSpec((_MQ, DM), row),
                   pl.BlockSpec((_MQ, 384), row),
                   pl.BlockSpec((_MQ, DM), row)],
        out_shape=[jax.ShapeDtypeStruct((N, K), jnp.int32),
                   jax.ShapeDtypeStruct((N, DM), jnp.float32),
                   jax.ShapeDtypeStruct((N, 384), jnp.int32),
                   jax.ShapeDtypeStruct((N, DM), jnp.float32)],
    )(xyzb, xyzTb, featb, fc1_w, fc1_b, wq, wk, wv, xyz3b, d1_w)


_NC, _NS = 2, 16
_NW = _NC * _NS
_TOT = BN * K
_HTOT = _TOT // 2
_PW = _HTOT // _NW
_CH = 128
_NCH = _PW // _CH


def _gather_body(idx_hbm, tab_hbm, out_hbm, idx_v, bufs, gsems, ssems):
    wid = lax.axis_index("s") * _NC + lax.axis_index("c")
    base = wid * _PW
    pltpu.sync_copy(idx_hbm.at[pl.ds(base, _PW)], idx_v)

    def start_gather(c, p):
        off = pl.multiple_of(c * _CH, _CH)
        ii = idx_v.at[pl.ds(off, _CH)]
        pltpu.async_copy(tab_hbm.at[ii], bufs[p], gsems[p])

    def wait_gather(p):
        pltpu.make_async_copy(tab_hbm.at[pl.ds(0, _CH)],
                              bufs[p], gsems[p]).wait()

    def start_scatter(c, p):
        gbase = pl.multiple_of(base + c * _CH, _CH)
        pltpu.async_copy(bufs[p], out_hbm.at[pl.ds(gbase, _CH)], ssems[p])

    def wait_scatter(p):
        pltpu.make_async_copy(tab_hbm.at[pl.ds(0, _CH)],
                              bufs[p], ssems[p]).wait()

    start_gather(0, 0)

    def body(i, carry):
        c0 = 2 * i
        wait_gather(0)

        @pl.when(i > 0)
        def _():
            wait_scatter(1)

        start_gather(c0 + 1, 1)
        start_scatter(c0, 0)
        wait_gather(1)
        wait_scatter(0)

        @pl.when(i < _NCH // 2 - 1)
        def _():
            start_gather(c0 + 2, 0)

        start_scatter(c0 + 1, 1)
        return carry

    lax.fori_loop(0, _NCH // 2, body, 0)
    wait_scatter(1)


def _sc_gather(idx_flat, tab):
    mesh = plsc.VectorSubcoreMesh(core_axis_name="c", subcore_axis_name="s")
    fn = pl.kernel(
        _gather_body,
        out_type=jax.ShapeDtypeStruct((_HTOT, 384), jnp.int32),
        mesh=mesh,
        scratch_types=[
            pltpu.VMEM((_PW,), jnp.int32),
            [pltpu.VMEM((_CH, 384), jnp.int32) for _ in range(2)],
            [pltpu.SemaphoreType.DMA for _ in range(2)],
            [pltpu.SemaphoreType.DMA for _ in range(2)],
        ],
    )
    return fn(idx_flat, tab)


_MB = 256
_MK = _MB * K


def _attn_body(q_ref, feat_ref, pq_ref, tab_ref,
               d1b_ref, d2_ref, d2b_ref, g1_ref, g1b_ref,
               g2_ref, g2b_ref, fc2_ref, fc2b_ref, res_ref, attn_ref):
    tab = tab_ref[...]
    kf_p, v_p, pg_p = tab[:, :128], tab[:, 128:256], tab[:, 256:384]
    pqb = pq_ref[...] + d1b_ref[...]
    pq_rep = jnp.reshape(
        jnp.broadcast_to(pqb.reshape(_MB, 1, DM), (_MB, K, DM)), (_MK, DM))
    pos1 = pq_rep - _unpack128(pg_p)
    h1 = jnp.maximum(pos1, 0.0)
    pos = jnp.dot(h1.astype(jnp.bfloat16), d2_ref[...].astype(jnp.bfloat16),
                  preferred_element_type=jnp.float32) + d2b_ref[...]

    qb = q_ref[...]
    qrep = jnp.reshape(
        jnp.broadcast_to(qb.reshape(_MB, 1, DM), (_MB, K, DM)), (_MK, DM))
    a_in = qrep - _unpack128(kf_p) + pos
    h2 = jnp.maximum(
        jnp.dot(a_in.astype(jnp.bfloat16), g1_ref[...].astype(jnp.bfloat16),
                preferred_element_type=jnp.float32)
        + g1b_ref[...], 0.0)
    t = jnp.dot(h2.astype(jnp.bfloat16), g2_ref[...].astype(jnp.bfloat16),
                preferred_element_type=jnp.float32) + g2b_ref[...]
    s3 = (t * (1.0 / 16.0)).reshape(_MB, K, DM)
    m = jnp.max(s3, axis=1, keepdims=True)
    e = jnp.exp(s3 - m)
    attn3 = e / jnp.sum(e, axis=1, keepdims=True)
    attn_ref[...] = attn3
    vp = (_unpack128(v_p) + pos).reshape(_MB, K, DM)
    r = jnp.sum(attn3 * vp, axis=1)
    res_ref[...] = (jnp.dot(r, fc2_ref[...], preferred_element_type=jnp.float32)
                    + fc2b_ref[...] + feat_ref[...])


_HB = BN // _MB // 2


def _attn_half(q, feat, pq, tab, d1_b, d2_w, d2_b,
               g1_w, g1_b, g2_w, g2_b, fc2_w, fc2_b, half, res_in, attn_in):
    full = lambda i: (0, 0)
    off = half * _HB
    row = lambda i: (i, 0)
    body = _attn_body
    in_specs = [
        pl.BlockSpec((_MB, DM), row),
        pl.BlockSpec((_MB, DP), row),
        pl.BlockSpec((_MB, DM), row),
        pl.BlockSpec((_MK, 384), lambda i: (i, 0)),
        pl.BlockSpec((1, DM), full),
        pl.BlockSpec((DM, DM), full),
        pl.BlockSpec((1, DM), full),
        pl.BlockSpec((DM, DM), full),
        pl.BlockSpec((1, DM), full),
        pl.BlockSpec((DM, DM), full),
        pl.BlockSpec((1, DM), full),
        pl.BlockSpec((DM, DP), full),
        pl.BlockSpec((1, DP), full),
    ]
    args = [q, feat, pq, tab, d1_b, d2_w, d2_b,
            g1_w, g1_b, g2_w, g2_b, fc2_w, fc2_b]
    aliases = {}
    if half:
        def body(q_ref, feat_ref, pq_ref, tab_ref, d1b_ref, d2_ref, d2b_ref,
                 g1_ref, g1b_ref, g2_ref, g2b_ref, fc2_ref, fc2b_ref,
                 ri_ref, ai_ref, res_ref, attn_ref):
            _attn_body(q_ref, feat_ref, pq_ref, tab_ref, d1b_ref, d2_ref,
                       d2b_ref, g1_ref, g1b_ref, g2_ref, g2b_ref, fc2_ref,
                       fc2b_ref, res_ref, attn_ref)
        in_specs += [pl.BlockSpec(memory_space=pl.ANY),
                     pl.BlockSpec(memory_space=pl.ANY)]
        args += [res_in, attn_in]
        aliases = {13: 0, 14: 1}
    return pl.pallas_call(
        body,
        grid=(_HB,),
        in_specs=in_specs,
        out_specs=[
            pl.BlockSpec((_MB, DP), lambda i: (i + off, 0)),
            pl.BlockSpec((_MB, K, DM), lambda i: (i + off, 0, 0)),
        ],
        out_shape=[
            jax.ShapeDtypeStruct((BN, DP), jnp.float32),
            jax.ShapeDtypeStruct((BN, K, DM), jnp.float32),
        ],
        input_output_aliases=aliases,
    )(*args)


def kernel(xyz, features, fc1_w, fc1_b, fc2_w, fc2_b, d1_w, d1_b, d2_w, d2_b,
           g1_w, g1_b, g2_w, g2_b, wq, wk, wv):
    feat = features.reshape(BN, DP)
    xyzT = jnp.transpose(xyz, (0, 2, 1))
    halves = []
    for b in range(B):
        xb = lax.slice(xyz, (b, 0, 0), (b + 1, N, 3))
        halves.append(_knn(
            xb, lax.slice(xyzT, (b, 0, 0), (b + 1, 3, N)),
            lax.slice(feat, (b * N, 0), ((b + 1) * N, DP)),
            fc1_w, fc1_b.reshape(1, DM), wq, wk, wv,
            xb.reshape(N, 3), d1_w))

    ws = (d1_b.reshape(1, DM), d2_w, d2_b.reshape(1, DM),
          g1_w, g1_b.reshape(1, DM), g2_w, g2_b.reshape(1, DM),
          fc2_w, fc2_b.reshape(1, DP))
    res, attn = None, None
    for b in range(B):
        idx_b, q_b, tab_b, pq_b = halves[b]
        gtab_b = _sc_gather(idx_b.reshape(_HTOT), tab_b)
        feat_b = lax.slice(feat, (b * N, 0), ((b + 1) * N, DP))
        res, attn = _attn_half(q_b, feat_b, pq_b, gtab_b, *ws, b, res, attn)
    return res.reshape(B, N, DP), attn.reshape(B, N, K, DM)


# speedup vs baseline: 1.0185x; 1.0185x over previous
import functools

import jax
import jax.numpy as jnp
from jax import lax
from jax.experimental import pallas as pl
from jax.experimental.pallas import tpu as pltpu
from jax.experimental.pallas import tpu_sc as plsc

B, N, K, DP, DM = 2, 2048, 16, 128, 256
BN = B * N

def _pack128(a):
    lo = a[:, :128].astype(jnp.bfloat16).astype(jnp.float32)
    hi = a[:, 128:].astype(jnp.bfloat16).astype(jnp.float32)
    lo_i = lax.bitcast_convert_type(lo, jnp.int32)
    hi_i = lax.bitcast_convert_type(hi, jnp.int32)
    return lax.shift_right_logical(lo_i, 16) | (hi_i & jnp.int32(-65536))


def _unpack128(xi):
    lo = lax.bitcast_convert_type(lax.shift_left(xi, 16), jnp.float32)
    hi = lax.bitcast_convert_type(xi & jnp.int32(-65536), jnp.float32)
    return jnp.concatenate([lo, hi], axis=1)


_MQ = 256


def _knn_body(xyz_ref, xyzT_ref, feat_ref, fc1w_ref, fc1b_ref,
              wq_ref, wk_ref, wv_ref, xyz3_ref, d1_ref,
              idx_ref, q_ref, tab_ref, p_ref):
    x = jnp.dot(feat_ref[...], fc1w_ref[...],
                preferred_element_type=jnp.float32) + fc1b_ref[...]
    xb = x.astype(jnp.bfloat16)
    q_ref[...] = jnp.dot(xb, wq_ref[...].astype(jnp.bfloat16),
                         preferred_element_type=jnp.float32)
    xk = jnp.dot(xb, wk_ref[...].astype(jnp.bfloat16),
                 preferred_element_type=jnp.float32)
    xv = jnp.dot(xb, wv_ref[...].astype(jnp.bfloat16),
                 preferred_element_type=jnp.float32)
    p = jnp.dot(xyz3_ref[...], d1_ref[...],
                preferred_element_type=jnp.float32)
    p_ref[...] = p
    tab_ref[...] = jnp.concatenate(
        [_pack128(xk), _pack128(xv), _pack128(p)], axis=1)
    xq = xyz_ref[0]
    xa = xyzT_ref[0]
    d0 = xq[:, 0:1] - xa[0:1, :]
    d1 = xq[:, 1:2] - xa[1:2, :]
    d2 = xq[:, 2:3] - xa[2:3, :]
    D = d0 * d0 + d1 * d1 + d2 * d2
    iota = lax.broadcasted_iota(jnp.int32, (_MQ, N), 1).astype(jnp.float32)
    cols = []
    for _ in range(K):
        mval = jnp.min(D, axis=1, keepdims=True)
        idxk = jnp.min(jnp.where(D == mval, iota, jnp.float32(N)),
                       axis=1, keepdims=True)
        cols.append(idxk)
        D = jnp.where(iota == idxk, jnp.float32(jnp.inf), D)
    idx_ref[...] = jnp.concatenate(cols, axis=1).astype(jnp.int32)


def _knn(xyzb, xyzTb, featb, fc1_w, fc1_b, wq, wk, wv, xyz3b, d1_w):
    nb = N // _MQ
    full = lambda m: (0, 0)
    row = lambda m: (m, 0)
    return pl.pallas_call(
        _knn_body,
        grid=(nb,),
        in_specs=[
            pl.BlockSpec((1, _MQ, 3), lambda m: (0, m, 0)),
            pl.BlockSpec((1, 3, N), lambda m: (0, 0, 0)),
            pl.BlockSpec((_MQ, DP), row),
            pl.BlockSpec((DP, DM), full),
            pl.BlockSpec((1, DM), full),
            pl.BlockSpec((DM, DM), full),
            pl.BlockSpec((DM, DM), full),
            pl.BlockSpec((DM, DM), full),
            pl.BlockSpec((_MQ, 3), row),
            pl.BlockSpec((3, DM), full),
        ],
        out_specs=[pl.BlockSpec((_MQ, K), row),
                   pl.BlockSpec((_MQ, DM), row),
                   pl.BlockSpec((_MQ, 384), row),
                   pl.BlockSpec((_MQ, DM), row)],
        out_shape=[jax.ShapeDtypeStruct((N, K), jnp.int32),
                   jax.ShapeDtypeStruct((N, DM), jnp.float32),
                   jax.ShapeDtypeStruct((N, 384), jnp.int32),
                   jax.ShapeDtypeStruct((N, DM), jnp.float32)],
    )(xyzb, xyzTb, featb, fc1_w, fc1_b, wq, wk, wv, xyz3b, d1_w)


_NC, _NS = 2, 16
_NW = _NC * _NS
_TOT = BN * K
_HTOT = _TOT // 2
_PW = _HTOT // _NW
_CH = 128
_NCH = _PW // _CH


def _gather_body(idx_hbm, tab_hbm, out_hbm, idx_v, bufs, gsems, ssems):
    wid = lax.axis_index("s") * _NC + lax.axis_index("c")
    base = wid * _PW
    pltpu.sync_copy(idx_hbm.at[pl.ds(base, _PW)], idx_v)

    def start_gather(c, p):
        off = pl.multiple_of(c * _CH, _CH)
        ii = idx_v.at[pl.ds(off, _CH)]
        pltpu.async_copy(tab_hbm.at[ii], bufs[p], gsems[p])

    def wait_gather(p):
        pltpu.make_async_copy(tab_hbm.at[pl.ds(0, _CH)],
                              bufs[p], gsems[p]).wait()

    def start_scatter(c, p):
        gbase = pl.multiple_of(base + c * _CH, _CH)
        pltpu.async_copy(bufs[p], out_hbm.at[pl.ds(gbase, _CH)], ssems[p])

    def wait_scatter(p):
        pltpu.make_async_copy(tab_hbm.at[pl.ds(0, _CH)],
                              bufs[p], ssems[p]).wait()

    start_gather(0, 0)

    def body(i, carry):
        c0 = 2 * i
        wait_gather(0)

        @pl.when(i > 0)
        def _():
            wait_scatter(1)

        start_gather(c0 + 1, 1)
        start_scatter(c0, 0)
        wait_gather(1)
        wait_scatter(0)

        @pl.when(i < _NCH // 2 - 1)
        def _():
            start_gather(c0 + 2, 0)

        start_scatter(c0 + 1, 1)
        return carry

    lax.fori_loop(0, _NCH // 2, body, 0)
    wait_scatter(1)


def _sc_gather(idx_flat, tab):
    mesh = plsc.VectorSubcoreMesh(core_axis_name="c", subcore_axis_name="s")
    fn = pl.kernel(
        _gather_body,
        out_type=jax.ShapeDtypeStruct((_HTOT, 384), jnp.int32),
        mesh=mesh,
        scratch_types=[
            pltpu.VMEM((_PW,), jnp.int32),
            [pltpu.VMEM((_CH, 384), jnp.int32) for _ in range(2)],
            [pltpu.SemaphoreType.DMA for _ in range(2)],
            [pltpu.SemaphoreType.DMA for _ in range(2)],
        ],
    )
    return fn(idx_flat, tab)


_MB = 256
_MK = _MB * K


def _attn_body(q_ref, feat_ref, pq_ref, tab_ref,
               d1b_ref, d2_ref, d2b_ref, g1_ref, g1b_ref,
               g2_ref, g2b_ref, fc2_ref, fc2b_ref, res_ref, attn_ref):
    tab = tab_ref[...]
    kf_p, v_p, pg_p = tab[:, :128], tab[:, 128:256], tab[:, 256:384]
    pqb = pq_ref[...] + d1b_ref[...]
    pq_rep = jnp.reshape(
        jnp.broadcast_to(pqb.reshape(_MB, 1, DM), (_MB, K, DM)), (_MK, DM))
    pos1 = pq_rep - _unpack128(pg_p)
    h1 = jnp.maximum(pos1, 0.0)
    pos = jnp.dot(h1.astype(jnp.bfloat16), d2_ref[...].astype(jnp.bfloat16),
                  preferred_element_type=jnp.float32) + d2b_ref[...]

    qb = q_ref[...]
    qrep = jnp.reshape(
        jnp.broadcast_to(qb.reshape(_MB, 1, DM), (_MB, K, DM)), (_MK, DM))
    a_in = qrep - _unpack128(kf_p) + pos
    h2 = jnp.maximum(
        jnp.dot(a_in.astype(jnp.bfloat16), g1_ref[...].astype(jnp.bfloat16),
                preferred_element_type=jnp.float32)
        + g1b_ref[...], 0.0)
    t = jnp.dot(h2.astype(jnp.bfloat16), g2_ref[...].astype(jnp.bfloat16),
                preferred_element_type=jnp.float32) + g2b_ref[...]
    s3 = (t * (1.0 / 16.0)).reshape(_MB, K, DM)
    m = jnp.max(s3, axis=1, keepdims=True)
    e = jnp.exp(s3 - m)
    attn3 = e / jnp.sum(e, axis=1, keepdims=True)
    attn_ref[...] = attn3
    vp = (_unpack128(v_p) + pos).reshape(_MB, K, DM)
    r = jnp.sum(attn3 * vp, axis=1)
    res_ref[...] = (jnp.dot(r, fc2_ref[...], preferred_element_type=jnp.float32)
                    + fc2b_ref[...] + feat_ref[...])


_HB = BN // _MB // 2


def _attn_half(q, feat, pq, tab, d1_b, d2_w, d2_b,
               g1_w, g1_b, g2_w, g2_b, fc2_w, fc2_b, half, res_in, attn_in):
    full = lambda i: (0, 0)
    off = half * _HB
    row = lambda i: (i, 0)
    body = _attn_body
    in_specs = [
        pl.BlockSpec((_MB, DM), row),
        pl.BlockSpec((_MB, DP), row),
        pl.BlockSpec((_MB, DM), row),
        pl.BlockSpec((_MK, 384), lambda i: (i, 0)),
        pl.BlockSpec((1, DM), full),
        pl.BlockSpec((DM, DM), full),
        pl.BlockSpec((1, DM), full),
        pl.BlockSpec((DM, DM), full),
        pl.BlockSpec((1, DM), full),
        pl.BlockSpec((DM, DM), full),
        pl.BlockSpec((1, DM), full),
        pl.BlockSpec((DM, DP), full),
        pl.BlockSpec((1, DP), full),
    ]
    args = [q, feat, pq, tab, d1_b, d2_w, d2_b,
            g1_w, g1_b, g2_w, g2_b, fc2_w, fc2_b]
    aliases = {}
    if half:
        def body(q_ref, feat_ref, pq_ref, tab_ref, d1b_ref, d2_ref, d2b_ref,
                 g1_ref, g1b_ref, g2_ref, g2b_ref, fc2_ref, fc2b_ref,
                 ri_ref, ai_ref, res_ref, attn_ref):
            _attn_body(q_ref, feat_ref, pq_ref, tab_ref, d1b_ref, d2_ref,
                       d2b_ref, g1_ref, g1b_ref, g2_ref, g2b_ref, fc2_ref,
                       fc2b_ref, res_ref, attn_ref)
        in_specs += [pl.BlockSpec(memory_space=pl.ANY),
                     pl.BlockSpec(memory_space=pl.ANY)]
        args += [res_in, attn_in]
        aliases = {13: 0, 14: 1}
    return pl.pallas_call(
        body,
        grid=(_HB,),
        in_specs=in_specs,
        out_specs=[
            pl.BlockSpec((_MB, DP), lambda i: (i + off, 0)),
            pl.BlockSpec((_MB, K, DM), lambda i: (i + off, 0, 0)),
        ],
        out_shape=[
            jax.ShapeDtypeStruct((BN, DP), jnp.float32),
            jax.ShapeDtypeStruct((BN, K, DM), jnp.float32),
        ],
        input_output_aliases=aliases,
    )(*args)


def kernel(xyz, features, fc1_w, fc1_b, fc2_w, fc2_b, d1_w, d1_b, d2_w, d2_b,
           g1_w, g1_b, g2_w, g2_b, wq, wk, wv):
    feat = features.reshape(BN, DP)
    xyzT = jnp.transpose(xyz, (0, 2, 1))
    halves = []
    for b in range(B):
        xb = lax.slice(xyz, (b, 0, 0), (b + 1, N, 3))
        halves.append(_knn(
            xb, lax.slice(xyzT, (b, 0, 0), (b + 1, 3, N)),
            lax.slice(feat, (b * N, 0), ((b + 1) * N, DP)),
            fc1_w, fc1_b.reshape(1, DM), wq, wk, wv,
            xb.reshape(N, 3), d1_w))

    ws = (d1_b.reshape(1, DM), d2_w, d2_b.reshape(1, DM),
          g1_w, g1_b.reshape(1, DM), g2_w, g2_b.reshape(1, DM),
          fc2_w, fc2_b.reshape(1, DP))
    res, attn = None, None
    for b in range(B):
        idx_b, q_b, tab_b, pq_b = halves[b]
        gtab_b = _sc_gather(idx_b.reshape(_HTOT), tab_b)
        feat_b = lax.slice(feat, (b * N, 0), ((b + 1) * N, DP))
        res, attn = _attn_half(q_b, feat_b, pq_b, gtab_b, *ws, b, res, attn)
    return res.reshape(B, N, DP), attn.reshape(B, N, K, DM)
